# baseline (speedup 1.0000x reference)
import functools

import jax
import jax.numpy as jnp
from jax.experimental import pallas as pl
from jax.experimental.pallas import tpu as pltpu
from jax.experimental.pallas import tpu_sc as plsc

_W = 256


def _tc_pad(table_t, vb=8192):
    dim, v = table_t.shape
    nmain = v // vb
    tail = v - nmain * vb
    nsteps = nmain + (1 if tail else 0)
    v_out = nsteps * vb

    def body(t_hbm, o_ref, buf, tbuf, sem, tsem):
        i = pl.program_id(0)
        slot = jax.lax.rem(i, 2)

        def main_copy(j, s):
            return pltpu.make_async_copy(
                t_hbm.at[:, pl.ds(j * vb, vb)], buf.at[s], sem.at[s]
            )

        tail_copy = pltpu.make_async_copy(
            t_hbm.at[:, pl.ds(nmain * vb, tail)], tbuf, tsem
        )

        @pl.when(i == 0)
        def _():
            main_copy(i, slot).start()

        @pl.when(i + 1 < nmain)
        def _():
            main_copy(i + 1, 1 - slot).start()

        @pl.when(tail and (i + 1 == nmain))
        def _():
            tail_copy.start()

        @pl.when(i < nmain)
        def _():
            main_copy(i, slot).wait()
            o_ref[:, :dim] = buf[slot].T

        if tail:
            @pl.when(i == nmain)
            def _():
                tail_copy.wait()
                o_ref[:tail, :dim] = tbuf[...].T

    return pl.pallas_call(
        body,
        out_shape=jax.ShapeDtypeStruct((v_out, 128), table_t.dtype),
        grid=(nsteps,),
        in_specs=[pl.BlockSpec(memory_space=pl.ANY)],
        out_specs=pl.BlockSpec((vb, 128), lambda i: (i, 0)),
        scratch_shapes=[
            pltpu.VMEM((2, dim, vb), table_t.dtype),
            pltpu.VMEM((dim, max(tail, 8)), table_t.dtype),
            pltpu.SemaphoreType.DMA((2,)),
            pltpu.SemaphoreType.DMA,
        ],
    )(table_t)


def kernel(token_ids, embed_matrix):
    batch, seq = token_ids.shape
    _, dim = embed_matrix.shape
    n = batch * seq
    idx = token_ids.reshape(1, n).astype(jnp.int32)
    table = _tc_pad(embed_matrix.T)

    mesh = plsc.VectorSubcoreMesh(core_axis_name="c", subcore_axis_name="s")

    @functools.partial(
        pl.kernel,
        out_type=jax.ShapeDtypeStruct((n, 128), embed_matrix.dtype),
        mesh=mesh,
    )
    def gather_kernel(table_hbm, idx_hbm, out_hbm):
        def body(i_vmem, o_vmem):
            pltpu.sync_copy(table_hbm.at[i_vmem.at[0]], o_vmem)

        pltpu.emit_pipeline(
            body,
            grid=(n // _W,),
            in_specs=[pl.BlockSpec((1, _W), lambda i: (0, i))],
            out_specs=[pl.BlockSpec((_W, 128), lambda i: (i, 0))],
            core_axis_name=("c", "s"),
            dimension_semantics=(pltpu.PARALLEL,),
        )(idx_hbm, out_hbm)

    out = gather_kernel(table, idx)
    return out.reshape(batch, seq, 128)[:, :, :dim]


# speedup vs baseline: 1.0240x; 1.0240x over previous
import functools

import jax
import jax.numpy as jnp
from jax.experimental import pallas as pl
from jax.experimental.pallas import tpu as pltpu
from jax.experimental.pallas import tpu_sc as plsc

_W = 256


def _tc_pad(table_t, vb=16384):
    dim, v = table_t.shape
    nmain = v // vb
    tail = v - nmain * vb
    nsteps = nmain + (1 if tail else 0)
    v_out = nsteps * vb

    def body(t_hbm, o_ref, buf, tbuf, sem, tsem):
        i = pl.program_id(0)
        slot = jax.lax.rem(i, 2)

        def main_copy(j, s):
            return pltpu.make_async_copy(
                t_hbm.at[:, pl.ds(j * vb, vb)], buf.at[s], sem.at[s]
            )

        tail_copy = pltpu.make_async_copy(
            t_hbm.at[:, pl.ds(nmain * vb, tail)], tbuf, tsem
        )

        @pl.when(i == 0)
        def _():
            main_copy(i, slot).start()

        @pl.when(i + 1 < nmain)
        def _():
            main_copy(i + 1, 1 - slot).start()

        @pl.when(tail and (i + 1 == nmain))
        def _():
            tail_copy.start()

        @pl.when(i < nmain)
        def _():
            main_copy(i, slot).wait()
            o_ref[:, :dim] = buf[slot].T

        if tail:
            @pl.when(i == nmain)
            def _():
                tail_copy.wait()
                o_ref[:tail, :dim] = tbuf[...].T

    return pl.pallas_call(
        body,
        out_shape=jax.ShapeDtypeStruct((v_out, 128), table_t.dtype),
        grid=(nsteps,),
        in_specs=[pl.BlockSpec(memory_space=pl.ANY)],
        out_specs=pl.BlockSpec((vb, 128), lambda i: (i, 0)),
        scratch_shapes=[
            pltpu.VMEM((2, dim, vb), table_t.dtype),
            pltpu.VMEM((dim, max(tail, 8)), table_t.dtype),
            pltpu.SemaphoreType.DMA((2,)),
            pltpu.SemaphoreType.DMA,
        ],
    )(table_t)


def kernel(token_ids, embed_matrix):
    batch, seq = token_ids.shape
    _, dim = embed_matrix.shape
    n = batch * seq
    idx = token_ids.reshape(1, n).astype(jnp.int32)
    table = _tc_pad(embed_matrix.T)

    mesh = plsc.VectorSubcoreMesh(core_axis_name="c", subcore_axis_name="s")

    @functools.partial(
        pl.kernel,
        out_type=jax.ShapeDtypeStruct((n, 128), embed_matrix.dtype),
        mesh=mesh,
    )
    def gather_kernel(table_hbm, idx_hbm, out_hbm):
        def body(i_vmem, o_vmem):
            pltpu.sync_copy(table_hbm.at[i_vmem.at[0]], o_vmem)

        pltpu.emit_pipeline(
            body,
            grid=(n // _W,),
            in_specs=[pl.BlockSpec((1, _W), lambda i: (0, i))],
            out_specs=[pl.BlockSpec((_W, 128), lambda i: (i, 0))],
            core_axis_name=("c", "s"),
            dimension_semantics=(pltpu.PARALLEL,),
        )(idx_hbm, out_hbm)

    out = gather_kernel(table, idx)
    return out.reshape(batch, seq, 128)[:, :, :dim]
